# baseline (speedup 1.0000x reference)
import jax
import jax.numpy as jnp
from jax.experimental import pallas as pl
from jax.experimental.pallas import tpu as pltpu

_ROWS_PER_BLOCK = 8


def _onehot_argmax_body(x_ref, o_ref):
    x = x_ref[...]
    v = x.shape[-1]
    m = jnp.max(x, axis=-1, keepdims=True)
    iota = jax.lax.broadcasted_iota(jnp.int32, x.shape, len(x.shape) - 1)
    masked = jnp.where(x == m, iota, jnp.int32(v))
    idx = jnp.min(masked, axis=-1, keepdims=True)
    o_ref[...] = (masked == idx).astype(jnp.float32)


def kernel(logits, temp, deterministic):
    b, g, v = logits.shape
    rows = b * g
    x2 = logits.reshape(rows, v)
    blk = _ROWS_PER_BLOCK if rows % _ROWS_PER_BLOCK == 0 else 1
    out = pl.pallas_call(
        _onehot_argmax_body,
        grid=(rows // blk,),
        in_specs=[pl.BlockSpec((blk, v), lambda i: (i, 0))],
        out_specs=pl.BlockSpec((blk, v), lambda i: (i, 0)),
        out_shape=jax.ShapeDtypeStruct((rows, v), jnp.float32),
        compiler_params=pltpu.CompilerParams(
            dimension_semantics=("parallel",),
        ),
    )(x2)
    return out.reshape(b, g, v)


# speedup vs baseline: 1.2775x; 1.2775x over previous
import jax
import jax.numpy as jnp
from jax.experimental import pallas as pl
from jax.experimental.pallas import tpu as pltpu

_ROWS_PER_BLOCK = 32


_SLICE_ROWS = 8


def _onehot_argmax_body(x_ref, o_ref):
    rows, v = x_ref.shape
    for r0 in range(0, rows, _SLICE_ROWS):
        x = x_ref[r0 : r0 + _SLICE_ROWS, :]
        m = jnp.max(x, axis=-1, keepdims=True)
        iota = jax.lax.broadcasted_iota(jnp.int32, x.shape, 1)
        masked = jnp.where(x == m, iota, jnp.int32(v))
        idx = jnp.min(masked, axis=-1, keepdims=True)
        o_ref[r0 : r0 + _SLICE_ROWS, :] = (masked == idx).astype(jnp.float32)


def kernel(logits, temp, deterministic):
    b, g, v = logits.shape
    rows = b * g
    x2 = logits.reshape(rows, v)
    blk = _ROWS_PER_BLOCK if rows % _ROWS_PER_BLOCK == 0 else 1
    out = pl.pallas_call(
        _onehot_argmax_body,
        grid=(rows // blk,),
        in_specs=[pl.BlockSpec((blk, v), lambda i: (i, 0))],
        out_specs=pl.BlockSpec((blk, v), lambda i: (i, 0)),
        out_shape=jax.ShapeDtypeStruct((rows, v), jnp.float32),
        compiler_params=pltpu.CompilerParams(
            dimension_semantics=("parallel",),
            vmem_limit_bytes=100 * 1024 * 1024,
        ),
    )(x2)
    return out.reshape(b, g, v)


# speedup vs baseline: 1.2776x; 1.0001x over previous
import jax
import jax.numpy as jnp
from jax.experimental import pallas as pl
from jax.experimental.pallas import tpu as pltpu

_ROWS_PER_BLOCK = 32


_SLICE_ROWS = 8


def _onehot_argmax_body(x_ref, o_ref):
    rows, v = x_ref.shape
    for r0 in range(0, rows, _SLICE_ROWS):
        x = x_ref[r0 : r0 + _SLICE_ROWS, :]
        m = jnp.max(x, axis=-1, keepdims=True)
        iota = jax.lax.broadcasted_iota(jnp.int32, x.shape, 1)
        masked = jnp.where(x == m, iota, jnp.int32(v))
        idx = jnp.min(masked, axis=-1, keepdims=True)
        o_ref[r0 : r0 + _SLICE_ROWS, :] = (masked == idx).astype(jnp.float32)


def kernel(logits, temp, deterministic):
    b, g, v = logits.shape
    rows = b * g
    x2 = logits.reshape(rows, v)
    blk = _ROWS_PER_BLOCK if rows % _ROWS_PER_BLOCK == 0 else 1
    out = pl.pallas_call(
        _onehot_argmax_body,
        grid=(rows // blk,),
        in_specs=[pl.BlockSpec((blk, v), lambda i: (i, 0))],
        out_specs=pl.BlockSpec((blk, v), lambda i: (i, 0)),
        out_shape=jax.ShapeDtypeStruct((rows, v), jnp.float32),
        compiler_params=pltpu.CompilerParams(
            dimension_semantics=("parallel",),
            vmem_limit_bytes=120 * 1024 * 1024,
        ),
    )(x2)
    return out.reshape(b, g, v)
